# Initial kernel scaffold; baseline (speedup 1.0000x reference)
import jax
import jax.numpy as jnp
from jax.experimental import pallas as pl


def kernel(x, edge_index, W, b):
    raise NotImplementedError("write your pallas kernel here")


# speedup vs baseline: 25.6180x; 25.6180x over previous
import functools

import jax
import jax.numpy as jnp
from jax import lax
from jax.experimental import pallas as pl
from jax.experimental.pallas import tpu as pltpu
from jax.experimental.pallas import tpu_sc as plsc

_ALPHA = 0.05
_BETA = 1.5

_N = 10000
_E = 320000
_D = 128
_NPAD = 10240
_NBLK = _NPAD // 128

_NC = 2
_NS = 16
_NW = _NC * _NS

_K1 = 79
_E1 = _NW * _K1 * 128
_C2 = 128
_K2 = 82
_E2 = _NW * _K2 * _C2
_PACK = 16384

_ROWS_PER_TILE = _NPAD // _NS

def _deg_body(rows_hbm, ones_hbm, degp_hbm, idx_v, ones_v, zer_v, acc_sh,
              dsem):
    c = lax.axis_index("c")
    s = lax.axis_index("s")
    wid = s * _NC + c
    pltpu.sync_copy(rows_hbm.at[wid], idx_v)
    pltpu.sync_copy(ones_hbm, ones_v)

    def zb(i, _):
        zer_v[pl.ds(i * 16, 16)] = jnp.zeros((16,), jnp.float32)
        return 0

    lax.fori_loop(0, _ROWS_PER_TILE // 16, zb, 0)
    base = s * _ROWS_PER_TILE
    pltpu.sync_copy(zer_v, acc_sh.at[pl.ds(base, _ROWS_PER_TILE)])
    plsc.subcore_barrier()

    def body(g, _):
        for t in range(8):
            j = g * 8 + t

            @pl.when(j < _K1)
            def _():
                pltpu.async_copy(ones_v, acc_sh.at[idx_v.at[j]], dsem,
                                 add=True)
        for t in range(8):
            j = g * 8 + t

            @pl.when(j < _K1)
            def _():
                pltpu.make_async_copy(ones_v, acc_sh.at[idx_v.at[j]],
                                      dsem).wait()
        return 0

    lax.fori_loop(0, (_K1 + 7) // 8, body, 0)
    plsc.subcore_barrier()
    pltpu.sync_copy(
        acc_sh.at[pl.ds(base, _ROWS_PER_TILE)],
        degp_hbm.at[c, pl.ds(base, _ROWS_PER_TILE)],
    )


def _spmm_body(pidx_hbm, hp_hbm, accp_hbm,
               pidx, uring, buf0, buf1, acc_sh,
               gsem0, gsem1, ssem0, ssem1):
    c = lax.axis_index("c")
    s = lax.axis_index("s")
    wid = s * _NC + c
    pltpu.sync_copy(pidx_hbm.at[wid], pidx)

    def unpack(j, phase):
        def ub(k, _):
            pk = pidx[j, pl.ds(k * 16, 16)]
            uring[phase, 0, pl.ds(k * 16, 16)] = lax.bitwise_and(pk, _PACK - 1)
            uring[phase, 1, pl.ds(k * 16, 16)] = lax.shift_right_logical(
                pk, 14)
            return 0
        lax.fori_loop(0, _C2 // 16, ub, 0)

    def zb(i, _):
        buf0[i // 8, pl.ds((i % 8) * 16, 16)] = jnp.zeros((16,), jnp.float32)
        return 0

    lax.fori_loop(0, _C2 * (_D // 16), zb, 0)
    base = s * _ROWS_PER_TILE
    for t in range(0, _ROWS_PER_TILE, _C2):
        n = min(_C2, _ROWS_PER_TILE - t)
        pltpu.sync_copy(buf0.at[pl.ds(0, n)], acc_sh.at[pl.ds(base + t, n)])
    plsc.subcore_barrier()

    bufs = (buf0, buf1)
    gsems = (gsem0, gsem1)
    ssems = (ssem0, ssem1)

    unpack(0, 0)
    pltpu.async_copy(hp_hbm.at[uring.at[0, 0]], buf0, gsem0)

    def step(j, tb, tp):
        nb, np_ = 1 - tb, (tp + 1) % 4

        @pl.when(j + 1 < _K2)
        def _():
            unpack(j + 1, np_)

        @pl.when(j >= 1)
        def _():
            pltpu.make_async_copy(
                bufs[nb], acc_sh.at[uring.at[(np_ + 2) % 4, 1]],
                ssems[nb]).wait()

        @pl.when(j + 1 < _K2)
        def _():
            pltpu.async_copy(hp_hbm.at[uring.at[np_, 0]], bufs[nb],
                             gsems[nb])

        pltpu.make_async_copy(hp_hbm.at[uring.at[tp, 0]], bufs[tb],
                              gsems[tb]).wait()
        pltpu.async_copy(bufs[tb], acc_sh.at[uring.at[tp, 1]], ssems[tb],
                         add=True)

    def body(i, _):
        j = 4 * i
        step(j, 0, 0)
        step(j + 1, 1, 1)
        step(j + 2, 0, 2)
        step(j + 3, 1, 3)
        return 0

    lax.fori_loop(0, _K2 // 4, body, 0)
    step(_K2 - 2, 0, (_K2 - 2) % 4)
    step(_K2 - 1, 1, (_K2 - 1) % 4)
    pltpu.make_async_copy(buf1, acc_sh.at[uring.at[(_K2 - 1) % 4, 1]],
                          ssem1).wait()
    plsc.subcore_barrier()
    for t in range(_ROWS_PER_TILE // 128):
        pltpu.sync_copy(
            acc_sh.at[pl.ds(base + t * 128, 128)],
            accp_hbm.at[c, pl.ds(base + t * 128, 128)],
        )


@functools.lru_cache(maxsize=None)
def _sc_kernels():
    mesh = plsc.VectorSubcoreMesh(core_axis_name="c", subcore_axis_name="s")
    deg_kernel = pl.kernel(
        _deg_body,
        out_type=jax.ShapeDtypeStruct((_NC, _NPAD), jnp.float32),
        mesh=mesh,
        scratch_types=[
            pltpu.VMEM((_K1, 128), jnp.int32),
            pltpu.VMEM((128,), jnp.float32),
            pltpu.VMEM((_ROWS_PER_TILE,), jnp.float32),
            pltpu.VMEM_SHARED((_NPAD,), jnp.float32),
            pltpu.SemaphoreType.DMA,
        ],
    )
    spmm_kernel = pl.kernel(
        _spmm_body,
        out_type=jax.ShapeDtypeStruct((_NC, _NPAD, _D), jnp.float32),
        mesh=mesh,
        scratch_types=[
            pltpu.VMEM((_K2, _C2), jnp.int32),
            pltpu.VMEM((4, 2, _C2), jnp.int32),
            pltpu.VMEM((_C2, _D), jnp.float32),
            pltpu.VMEM((_C2, _D), jnp.float32),
            pltpu.VMEM_SHARED((_NPAD, _D), jnp.float32),
            pltpu.SemaphoreType.DMA,
            pltpu.SemaphoreType.DMA,
            pltpu.SemaphoreType.DMA,
            pltpu.SemaphoreType.DMA,
        ],
    )
    return deg_kernel, spmm_kernel


def _mm_body(x_ref, w_ref, b_ref, h_ref):
    x = x_ref[...]
    w = w_ref[...]
    h = lax.dot_general(x, w, (((1,), (1,)), ((), ())),
                        preferred_element_type=jnp.float32)
    h_ref[...] = h + b_ref[0, :][None, :]


def _mm_call(xp, W, bb):
    return pl.pallas_call(
        _mm_body,
        grid=(_NBLK,),
        in_specs=[
            pl.BlockSpec((128, _D), lambda i: (i, 0)),
            pl.BlockSpec((_D, _D), lambda i: (0, 0)),
            pl.BlockSpec((8, _D), lambda i: (0, 0)),
        ],
        out_specs=pl.BlockSpec((128, _D), lambda i: (i, 0)),
        out_shape=jax.ShapeDtypeStruct((_NPAD, _D), jnp.float32),
    )(xp, W, bb)


def _scale_body(h_ref, d0_ref, d1_ref, hp_ref, dis_ref):
    h = h_ref[...]
    deg = 1.0 + d0_ref[0, 0, :] + d1_ref[0, 0, :]
    dis = lax.rsqrt(deg)
    dis_ref[0, 0, :] = dis
    hp_ref[...] = h * dis[:, None]


def _scale_call(h, d0, d1):
    return pl.pallas_call(
        _scale_body,
        grid=(_NBLK,),
        in_specs=[
            pl.BlockSpec((128, _D), lambda i: (i, 0)),
            pl.BlockSpec((1, 1, 128), lambda i: (i, 0, 0)),
            pl.BlockSpec((1, 1, 128), lambda i: (i, 0, 0)),
        ],
        out_specs=[
            pl.BlockSpec((128, _D), lambda i: (i, 0)),
            pl.BlockSpec((1, 1, 128), lambda i: (i, 0, 0)),
        ],
        out_shape=[
            jax.ShapeDtypeStruct((_NPAD, _D), jnp.float32),
            jax.ShapeDtypeStruct((_NBLK, 1, 128), jnp.float32),
        ],
    )(h, d0, d1)


def _comb_body(h_ref, a0_ref, a1_ref, dis_ref, o_ref):
    h = h_ref[...]
    a = a0_ref[...] + a1_ref[...]
    dis = dis_ref[0, 0, :]
    o_ref[...] = ((1.0 - _ALPHA) * h + _ALPHA * (dis[:, None] * a)
                  + _BETA * jnp.maximum(h, 0.0))


def _comb_call(h, a0, a1, dis):
    return pl.pallas_call(
        _comb_body,
        grid=(_NBLK,),
        in_specs=[
            pl.BlockSpec((128, _D), lambda i: (i, 0)),
            pl.BlockSpec((128, _D), lambda i: (i, 0)),
            pl.BlockSpec((128, _D), lambda i: (i, 0)),
            pl.BlockSpec((1, 1, 128), lambda i: (i, 0, 0)),
        ],
        out_specs=pl.BlockSpec((128, _D), lambda i: (i, 0)),
        out_shape=jax.ShapeDtypeStruct((_NPAD, _D), jnp.float32),
    )(h, a0, a1, dis)


@jax.jit
def kernel(x, edge_index, W, b):
    row = edge_index[0]
    col = edge_index[1]
    loops = jnp.arange(_N, dtype=jnp.int32)

    pad1 = _N + jnp.arange(_E1 - _E, dtype=jnp.int32) % (_NPAD - _N)
    rows1 = jnp.concatenate([row, pad1]).reshape(_NW, _K1, 128)
    npad2 = _E2 - _E - _N
    padr = _N + jnp.arange(npad2, dtype=jnp.int32) % (_NPAD - _N)
    padc = jnp.arange(npad2, dtype=jnp.int32) % _N
    packed = jnp.concatenate(
        [row * _PACK + col, loops * _PACK + loops, padr * _PACK + padc]
    ).reshape(_NW, _K2, _C2)
    ones_v = jnp.ones((128,), jnp.float32)

    deg_kernel, spmm_kernel = _sc_kernels()
    degp = deg_kernel(rows1, ones_v)

    xp = jnp.concatenate([x, jnp.zeros((_NPAD - _N, _D), jnp.float32)])
    bb = jnp.broadcast_to(b, (8, _D))
    h = _mm_call(xp, W, bb)
    d0 = degp[0].reshape(_NBLK, 1, 128)
    d1 = degp[1].reshape(_NBLK, 1, 128)
    hp, dis = _scale_call(h, d0, d1)

    accp = spmm_kernel(packed, hp)
    out = _comb_call(h, accp[0], accp[1], dis)
    return out[:_N]


# speedup vs baseline: 30.1286x; 1.1761x over previous
import functools

import jax
import jax.numpy as jnp
from jax import lax
from jax.experimental import pallas as pl
from jax.experimental.pallas import tpu as pltpu
from jax.experimental.pallas import tpu_sc as plsc

_ALPHA = 0.05
_BETA = 1.5

_N = 10000
_E = 320000
_D = 128
_NPAD = 10240
_NBLK = _NPAD // 128

_NC = 2
_NS = 16
_NW = _NC * _NS

_K1 = 79
_E1 = _NW * _K1 * 128
_C2 = 128
_K2 = 82
_E2 = _NW * _K2 * _C2
_PACK = 16384

_ROWS_PER_TILE = _NPAD // _NS

def _deg_body(rows_hbm, ones_hbm, degp_hbm, idx_v, ones_v, zer_v, acc_sh,
              dsem):
    c = lax.axis_index("c")
    s = lax.axis_index("s")
    wid = s * _NC + c
    pltpu.sync_copy(rows_hbm.at[wid], idx_v)
    pltpu.sync_copy(ones_hbm, ones_v)

    def zb(i, _):
        zer_v[pl.ds(i * 16, 16)] = jnp.zeros((16,), jnp.float32)
        return 0

    lax.fori_loop(0, _ROWS_PER_TILE // 16, zb, 0)
    base = s * _ROWS_PER_TILE
    pltpu.sync_copy(zer_v, acc_sh.at[pl.ds(base, _ROWS_PER_TILE)])
    plsc.subcore_barrier()

    def body(g, _):
        for t in range(8):
            j = g * 8 + t

            @pl.when(j < _K1)
            def _():
                pltpu.async_copy(ones_v, acc_sh.at[idx_v.at[j]], dsem,
                                 add=True)
        for t in range(8):
            j = g * 8 + t

            @pl.when(j < _K1)
            def _():
                pltpu.make_async_copy(ones_v, acc_sh.at[idx_v.at[j]],
                                      dsem).wait()
        return 0

    lax.fori_loop(0, (_K1 + 7) // 8, body, 0)
    plsc.subcore_barrier()
    pltpu.sync_copy(
        acc_sh.at[pl.ds(base, _ROWS_PER_TILE)],
        degp_hbm.at[c, pl.ds(base, _ROWS_PER_TILE)],
    )


def _spmm_body(pidx_hbm, hp_hbm, accp_hbm,
               pidx, uring, buf0, buf1, acc_sh,
               gsem0, gsem1, ssem0, ssem1):
    c = lax.axis_index("c")
    s = lax.axis_index("s")
    wid = s * _NC + c
    pltpu.sync_copy(pidx_hbm.at[wid], pidx)

    def unpack(j, phase):
        def ub(k, _):
            pk = pidx[j, pl.ds(k * 16, 16)]
            uring[phase, 0, pl.ds(k * 16, 16)] = lax.bitwise_and(pk, _PACK - 1)
            uring[phase, 1, pl.ds(k * 16, 16)] = lax.shift_right_logical(
                pk, 14)
            return 0
        lax.fori_loop(0, _C2 // 16, ub, 0)

    def zb(i, _):
        buf0[i // 8, pl.ds((i % 8) * 16, 16)] = jnp.zeros((16,), jnp.float32)
        return 0

    lax.fori_loop(0, _C2 * (_D // 16), zb, 0)
    base = s * _ROWS_PER_TILE
    for t in range(0, _ROWS_PER_TILE, _C2):
        n = min(_C2, _ROWS_PER_TILE - t)
        pltpu.sync_copy(buf0.at[pl.ds(0, n)], acc_sh.at[pl.ds(base + t, n)])
    plsc.subcore_barrier()

    bufs = (buf0, buf1)
    gsems = (gsem0, gsem1)
    ssems = (ssem0, ssem1)

    unpack(0, 0)
    pltpu.async_copy(hp_hbm.at[uring.at[0, 0]], buf0, gsem0)

    def step(j, tb, tp):
        nb, np_ = 1 - tb, (tp + 1) % 4

        @pl.when(j + 1 < _K2)
        def _():
            unpack(j + 1, np_)

        @pl.when(j >= 1)
        def _():
            pltpu.make_async_copy(
                bufs[nb], acc_sh.at[uring.at[(np_ + 2) % 4, 1]],
                ssems[nb]).wait()

        @pl.when(j + 1 < _K2)
        def _():
            pltpu.async_copy(hp_hbm.at[uring.at[np_, 0]], bufs[nb],
                             gsems[nb])

        pltpu.make_async_copy(hp_hbm.at[uring.at[tp, 0]], bufs[tb],
                              gsems[tb]).wait()
        pltpu.async_copy(bufs[tb], acc_sh.at[uring.at[tp, 1]], ssems[tb],
                         add=True)

    def body(i, _):
        j = 4 * i
        step(j, 0, 0)
        step(j + 1, 1, 1)
        step(j + 2, 0, 2)
        step(j + 3, 1, 3)
        return 0

    lax.fori_loop(0, _K2 // 4, body, 0)
    step(_K2 - 2, 0, (_K2 - 2) % 4)
    step(_K2 - 1, 1, (_K2 - 1) % 4)
    pltpu.make_async_copy(buf1, acc_sh.at[uring.at[(_K2 - 1) % 4, 1]],
                          ssem1).wait()
    plsc.subcore_barrier()
    for t in range(_ROWS_PER_TILE // 128):
        pltpu.sync_copy(
            acc_sh.at[pl.ds(base + t * 128, 128)],
            accp_hbm.at[c, pl.ds(base + t * 128, 128)],
        )


@functools.lru_cache(maxsize=None)
def _sc_kernels():
    mesh = plsc.VectorSubcoreMesh(core_axis_name="c", subcore_axis_name="s")
    deg_kernel = pl.kernel(
        _deg_body,
        out_type=jax.ShapeDtypeStruct((_NC, _NPAD), jnp.float32),
        mesh=mesh,
        scratch_types=[
            pltpu.VMEM((_K1, 128), jnp.int32),
            pltpu.VMEM((128,), jnp.float32),
            pltpu.VMEM((_ROWS_PER_TILE,), jnp.float32),
            pltpu.VMEM_SHARED((_NPAD,), jnp.float32),
            pltpu.SemaphoreType.DMA,
        ],
    )
    spmm_kernel = pl.kernel(
        _spmm_body,
        out_type=jax.ShapeDtypeStruct((_NC, _NPAD, _D), jnp.float32),
        mesh=mesh,
        scratch_types=[
            pltpu.VMEM((_K2, _C2), jnp.int32),
            pltpu.VMEM((4, 2, _C2), jnp.int32),
            pltpu.VMEM((_C2, _D), jnp.float32),
            pltpu.VMEM((_C2, _D), jnp.float32),
            pltpu.VMEM_SHARED((_NPAD, _D), jnp.float32),
            pltpu.SemaphoreType.DMA,
            pltpu.SemaphoreType.DMA,
            pltpu.SemaphoreType.DMA,
            pltpu.SemaphoreType.DMA,
        ],
    )
    return deg_kernel, spmm_kernel


def _mm_body(x_ref, w_ref, b_ref, d0_ref, d1_ref, h_ref, hp_ref, dis_ref):
    x = x_ref[...]
    w = w_ref[...]
    h = lax.dot_general(x, w, (((1,), (1,)), ((), ())),
                        preferred_element_type=jnp.float32)
    h = h + b_ref[0, :][None, :]
    deg = 1.0 + d0_ref[0, 0, :] + d1_ref[0, 0, :]
    dis = lax.rsqrt(deg)
    dis_ref[0, 0, :] = dis
    h_ref[...] = h
    hp_ref[...] = h * dis[:, None]


def _mm_call(xp, W, bb, d0, d1):
    return pl.pallas_call(
        _mm_body,
        grid=(_NBLK,),
        in_specs=[
            pl.BlockSpec((128, _D), lambda i: (i, 0)),
            pl.BlockSpec((_D, _D), lambda i: (0, 0)),
            pl.BlockSpec((8, _D), lambda i: (0, 0)),
            pl.BlockSpec((1, 1, 128), lambda i: (i, 0, 0)),
            pl.BlockSpec((1, 1, 128), lambda i: (i, 0, 0)),
        ],
        out_specs=[
            pl.BlockSpec((128, _D), lambda i: (i, 0)),
            pl.BlockSpec((128, _D), lambda i: (i, 0)),
            pl.BlockSpec((1, 1, 128), lambda i: (i, 0, 0)),
        ],
        out_shape=[
            jax.ShapeDtypeStruct((_NPAD, _D), jnp.float32),
            jax.ShapeDtypeStruct((_NPAD, _D), jnp.float32),
            jax.ShapeDtypeStruct((_NBLK, 1, 128), jnp.float32),
        ],
    )(xp, W, bb, d0, d1)


def _comb_body(h_ref, a0_ref, a1_ref, dis_ref, o_ref):
    h = h_ref[...]
    a = a0_ref[...] + a1_ref[...]
    dis = dis_ref[0, 0, :]
    o_ref[...] = ((1.0 - _ALPHA) * h + _ALPHA * (dis[:, None] * a)
                  + _BETA * jnp.maximum(h, 0.0))


def _comb_call(h, a0, a1, dis):
    return pl.pallas_call(
        _comb_body,
        grid=(_NBLK,),
        in_specs=[
            pl.BlockSpec((128, _D), lambda i: (i, 0)),
            pl.BlockSpec((128, _D), lambda i: (i, 0)),
            pl.BlockSpec((128, _D), lambda i: (i, 0)),
            pl.BlockSpec((1, 1, 128), lambda i: (i, 0, 0)),
        ],
        out_specs=pl.BlockSpec((128, _D), lambda i: (i, 0)),
        out_shape=jax.ShapeDtypeStruct((_NPAD, _D), jnp.float32),
    )(h, a0, a1, dis)


@jax.jit
def kernel(x, edge_index, W, b):
    row = edge_index[0]
    col = edge_index[1]
    loops = jnp.arange(_N, dtype=jnp.int32)

    pad1 = _N + jnp.arange(_E1 - _E, dtype=jnp.int32) % (_NPAD - _N)
    rows1 = jnp.concatenate([row, pad1]).reshape(_NW, _K1, 128)
    npad2 = _E2 - _E - _N
    padr = _N + jnp.arange(npad2, dtype=jnp.int32) % (_NPAD - _N)
    padc = jnp.arange(npad2, dtype=jnp.int32) % _N
    packed = jnp.concatenate(
        [row * _PACK + col, loops * _PACK + loops, padr * _PACK + padc]
    ).reshape(_NW, _K2, _C2)
    ones_v = jnp.ones((128,), jnp.float32)

    deg_kernel, spmm_kernel = _sc_kernels()
    degp = deg_kernel(rows1, ones_v)

    xp = jnp.concatenate([x, jnp.zeros((_NPAD - _N, _D), jnp.float32)])
    bb = jnp.broadcast_to(b, (8, _D))
    d0 = degp[0].reshape(_NBLK, 1, 128)
    d1 = degp[1].reshape(_NBLK, 1, 128)
    h, hp, dis = _mm_call(xp, W, bb, d0, d1)

    accp = spmm_kernel(packed, hp)
    out = _comb_call(h, accp[0], accp[1], dis)
    return out[:_N]


# speedup vs baseline: 30.1623x; 1.0011x over previous
import functools

import jax
import jax.numpy as jnp
from jax import lax
from jax.experimental import pallas as pl
from jax.experimental.pallas import tpu as pltpu
from jax.experimental.pallas import tpu_sc as plsc

_ALPHA = 0.05
_BETA = 1.5

_N = 10000
_E = 320000
_D = 128
_NPAD = 10240
_NBLK = _NPAD // 128

_NC = 2
_NS = 16
_NW = _NC * _NS

_K1 = 79
_E1 = _NW * _K1 * 128
_C2 = 128
_K2 = 80
_E2 = _NW * _K2 * _C2
_PACK = 16384

_ROWS_PER_TILE = _NPAD // _NS

def _deg_body(rows_hbm, ones_hbm, degp_hbm, idx_v, ones_v, zer_v, acc_sh,
              dsem):
    c = lax.axis_index("c")
    s = lax.axis_index("s")
    wid = s * _NC + c
    pltpu.sync_copy(rows_hbm.at[wid], idx_v)
    pltpu.sync_copy(ones_hbm, ones_v)

    def zb(i, _):
        zer_v[pl.ds(i * 16, 16)] = jnp.zeros((16,), jnp.float32)
        return 0

    lax.fori_loop(0, _ROWS_PER_TILE // 16, zb, 0)
    base = s * _ROWS_PER_TILE
    pltpu.sync_copy(zer_v, acc_sh.at[pl.ds(base, _ROWS_PER_TILE)])
    plsc.subcore_barrier()

    def body(g, _):
        for t in range(8):
            j = g * 8 + t

            @pl.when(j < _K1)
            def _():
                pltpu.async_copy(ones_v, acc_sh.at[idx_v.at[j]], dsem,
                                 add=True)
        for t in range(8):
            j = g * 8 + t

            @pl.when(j < _K1)
            def _():
                pltpu.make_async_copy(ones_v, acc_sh.at[idx_v.at[j]],
                                      dsem).wait()
        return 0

    lax.fori_loop(0, (_K1 + 7) // 8, body, 0)
    plsc.subcore_barrier()
    pltpu.sync_copy(
        acc_sh.at[pl.ds(base, _ROWS_PER_TILE)],
        degp_hbm.at[c, pl.ds(base, _ROWS_PER_TILE)],
    )


def _spmm_body(pidx_hbm, hp_hbm, accp_hbm,
               pidx, uring, buf0, buf1, acc_sh,
               gsem0, gsem1, ssem0, ssem1):
    c = lax.axis_index("c")
    s = lax.axis_index("s")
    wid = s * _NC + c
    pltpu.sync_copy(pidx_hbm.at[wid], pidx)
    base = s * _ROWS_PER_TILE

    def unpack(j, phase):
        def ub(k, _):
            pk = pidx[j, pl.ds(k * 16, 16)]
            uring[phase, 0, pl.ds(k * 16, 16)] = lax.bitwise_and(pk, _PACK - 1)
            uring[phase, 1, pl.ds(k * 16, 16)] = lax.shift_right_logical(
                pk, 14)
            return 0
        lax.fori_loop(0, _C2 // 16, ub, 0)

    nslice = _ROWS_PER_TILE // 128
    ibufs = (buf0, buf1)
    isems = (gsem0, gsem1)
    pltpu.async_copy(hp_hbm.at[pl.ds(base, 128)], buf0, gsem0)
    for t in range(nslice):
        pltpu.make_async_copy(hp_hbm.at[pl.ds(base + t * 128, 128)],
                              ibufs[t % 2], isems[t % 2]).wait()
        if t + 1 < nslice:
            pltpu.async_copy(hp_hbm.at[pl.ds(base + (t + 1) * 128, 128)],
                             ibufs[(t + 1) % 2], isems[(t + 1) % 2])
        pltpu.sync_copy(ibufs[t % 2], acc_sh.at[pl.ds(base + t * 128, 128)])
    plsc.subcore_barrier()

    bufs = (buf0, buf1)
    gsems = (gsem0, gsem1)
    ssems = (ssem0, ssem1)

    unpack(0, 0)
    pltpu.async_copy(hp_hbm.at[uring.at[0, 0]], buf0, gsem0)

    def step(j, tb, tp):
        nb, np_ = 1 - tb, (tp + 1) % 4

        @pl.when(j + 1 < _K2)
        def _():
            unpack(j + 1, np_)

        @pl.when(j >= 1)
        def _():
            pltpu.make_async_copy(
                bufs[nb], acc_sh.at[uring.at[(np_ + 2) % 4, 1]],
                ssems[nb]).wait()

        @pl.when(j + 1 < _K2)
        def _():
            pltpu.async_copy(hp_hbm.at[uring.at[np_, 0]], bufs[nb],
                             gsems[nb])

        pltpu.make_async_copy(hp_hbm.at[uring.at[tp, 0]], bufs[tb],
                              gsems[tb]).wait()
        pltpu.async_copy(bufs[tb], acc_sh.at[uring.at[tp, 1]], ssems[tb],
                         add=True)

    def body(i, _):
        j = 4 * i
        step(j, 0, 0)
        step(j + 1, 1, 1)
        step(j + 2, 0, 2)
        step(j + 3, 1, 3)
        return 0

    lax.fori_loop(0, _K2 // 4, body, 0)
    pltpu.make_async_copy(buf1, acc_sh.at[uring.at[(_K2 - 1) % 4, 1]],
                          ssem1).wait()
    plsc.subcore_barrier()
    for t in range(_ROWS_PER_TILE // 128):
        pltpu.sync_copy(
            acc_sh.at[pl.ds(base + t * 128, 128)],
            accp_hbm.at[c, pl.ds(base + t * 128, 128)])


@functools.lru_cache(maxsize=None)
def _sc_kernels():
    mesh = plsc.VectorSubcoreMesh(core_axis_name="c", subcore_axis_name="s")
    deg_kernel = pl.kernel(
        _deg_body,
        out_type=jax.ShapeDtypeStruct((_NC, _NPAD), jnp.float32),
        mesh=mesh,
        scratch_types=[
            pltpu.VMEM((_K1, 128), jnp.int32),
            pltpu.VMEM((128,), jnp.float32),
            pltpu.VMEM((_ROWS_PER_TILE,), jnp.float32),
            pltpu.VMEM_SHARED((_NPAD,), jnp.float32),
            pltpu.SemaphoreType.DMA,
        ],
    )
    spmm_kernel = pl.kernel(
        _spmm_body,
        out_type=jax.ShapeDtypeStruct((_NC, _NPAD, _D), jnp.float32),
        mesh=mesh,
        scratch_types=[
            pltpu.VMEM((_K2, _C2), jnp.int32),
            pltpu.VMEM((4, 2, _C2), jnp.int32),
            pltpu.VMEM((_C2, _D), jnp.float32),
            pltpu.VMEM((_C2, _D), jnp.float32),
            pltpu.VMEM_SHARED((_NPAD, _D), jnp.float32),
            pltpu.SemaphoreType.DMA,
            pltpu.SemaphoreType.DMA,
            pltpu.SemaphoreType.DMA,
            pltpu.SemaphoreType.DMA,
        ],
    )
    return deg_kernel, spmm_kernel


def _mm_body(x_ref, w_ref, b_ref, d0_ref, d1_ref, h_ref, hp_ref, dis_ref):
    x = x_ref[...]
    w = w_ref[...]
    h = lax.dot_general(x, w, (((1,), (1,)), ((), ())),
                        preferred_element_type=jnp.float32)
    h = h + b_ref[0, :][None, :]
    deg = 1.0 + d0_ref[0, 0, :] + d1_ref[0, 0, :]
    dis = lax.rsqrt(deg)
    dis_ref[0, 0, :] = dis
    h_ref[...] = h
    hp_ref[...] = h * dis[:, None]


def _mm_call(xp, W, bb, d0, d1):
    return pl.pallas_call(
        _mm_body,
        grid=(_NBLK,),
        in_specs=[
            pl.BlockSpec((128, _D), lambda i: (i, 0)),
            pl.BlockSpec((_D, _D), lambda i: (0, 0)),
            pl.BlockSpec((8, _D), lambda i: (0, 0)),
            pl.BlockSpec((1, 1, 128), lambda i: (i, 0, 0)),
            pl.BlockSpec((1, 1, 128), lambda i: (i, 0, 0)),
        ],
        out_specs=[
            pl.BlockSpec((128, _D), lambda i: (i, 0)),
            pl.BlockSpec((128, _D), lambda i: (i, 0)),
            pl.BlockSpec((1, 1, 128), lambda i: (i, 0, 0)),
        ],
        out_shape=[
            jax.ShapeDtypeStruct((_NPAD, _D), jnp.float32),
            jax.ShapeDtypeStruct((_NPAD, _D), jnp.float32),
            jax.ShapeDtypeStruct((_NBLK, 1, 128), jnp.float32),
        ],
    )(xp, W, bb, d0, d1)


def _comb_body(h_ref, a0_ref, a1_ref, dis_ref, o_ref):
    h = h_ref[...]
    a = a0_ref[...] + a1_ref[...]
    dis = dis_ref[0, 0, :]
    o_ref[...] = ((1.0 - _ALPHA) * h + _ALPHA * (dis[:, None] * a)
                  + _BETA * jnp.maximum(h, 0.0))


def _comb_call(h, a0, a1, dis):
    return pl.pallas_call(
        _comb_body,
        grid=(_NBLK,),
        in_specs=[
            pl.BlockSpec((128, _D), lambda i: (i, 0)),
            pl.BlockSpec((128, _D), lambda i: (i, 0)),
            pl.BlockSpec((128, _D), lambda i: (i, 0)),
            pl.BlockSpec((1, 1, 128), lambda i: (i, 0, 0)),
        ],
        out_specs=pl.BlockSpec((128, _D), lambda i: (i, 0)),
        out_shape=jax.ShapeDtypeStruct((_NPAD, _D), jnp.float32),
    )(h, a0, a1, dis)


@jax.jit
def kernel(x, edge_index, W, b):
    row = edge_index[0]
    col = edge_index[1]

    pad1 = _N + jnp.arange(_E1 - _E, dtype=jnp.int32) % (_NPAD - _N)
    rows1 = jnp.concatenate([row, pad1]).reshape(_NW, _K1, 128)
    npad2 = _E2 - _E
    padr = _N + jnp.arange(npad2, dtype=jnp.int32) % (_NPAD - _N)
    padc = jnp.arange(npad2, dtype=jnp.int32) % _N
    packed = jnp.concatenate(
        [row * _PACK + col, padr * _PACK + padc]
    ).reshape(_NW, _K2, _C2)
    ones_v = jnp.ones((128,), jnp.float32)

    deg_kernel, spmm_kernel = _sc_kernels()
    degp = deg_kernel(rows1, ones_v)

    xp = jnp.concatenate([x, jnp.zeros((_NPAD - _N, _D), jnp.float32)])
    bb = jnp.broadcast_to(b, (8, _D))
    d0 = degp[0].reshape(_NBLK, 1, 128)
    d1 = degp[1].reshape(_NBLK, 1, 128)
    h, hp, dis = _mm_call(xp, W, bb, d0, d1)

    accp = spmm_kernel(packed, hp)
    out = _comb_call(h, accp[0], accp[1], dis)
    return out[:_N]
